# Initial kernel scaffold; baseline (speedup 1.0000x reference)
import jax
import jax.numpy as jnp
from jax.experimental import pallas as pl


def kernel(input, table, W, b):
    raise NotImplementedError("write your pallas kernel here")


# speedup vs baseline: 2.7059x; 2.7059x over previous
import jax
import jax.numpy as jnp
from jax import lax
from jax.experimental import pallas as pl
from jax.experimental.pallas import tpu as pltpu
from jax.experimental.pallas import tpu_sc as plsc

NUM_CORES = 2
NUM_SUBCORES = 16
NW = NUM_CORES * NUM_SUBCORES

BATCH = 16384
HIST = 50
EMBED = 32
OUT_DIM = 20

BAGS_PER_W = BATCH // NW
G_BAGS = 32
G_ROWS = G_BAGS * HIST
N_GROUPS = BAGS_PER_W // G_BAGS
CHUNK = 64
N_CHUNKS = G_ROWS // CHUNK


def _sc_body(idx_hbm, table_hbm, out_hbm, idx_v, rows_v, out_v, gsem):
    wid = lax.axis_index("s") * NUM_CORES + lax.axis_index("c")
    bag0 = wid * BAGS_PER_W

    def group(g, carry):
        row0 = (bag0 + g * G_BAGS) * HIST
        pltpu.sync_copy(idx_hbm.at[pl.ds(row0, G_ROWS)], idx_v)
        handles = []
        for c in range(N_CHUNKS):
            handles.append(pltpu.async_copy(
                table_hbm.at[idx_v.at[pl.ds(c * CHUNK, CHUNK)]],
                rows_v.at[pl.ds(c * CHUNK, CHUNK)],
                gsem))
        for h in handles:
            h.wait()

        def bag(i, c):
            r0 = i * HIST
            acc0 = jnp.zeros((16,), jnp.float32)
            acc1 = jnp.zeros((16,), jnp.float32)
            for r in range(HIST):
                acc0 = acc0 + rows_v[r0 + r, pl.ds(0, 16)]
                acc1 = acc1 + rows_v[r0 + r, pl.ds(16, 16)]
            out_v[i, pl.ds(0, 16)] = acc0 * (1.0 / HIST)
            out_v[i, pl.ds(16, 16)] = acc1 * (1.0 / HIST)
            return c

        lax.fori_loop(0, G_BAGS, bag, 0)
        pltpu.sync_copy(out_v, out_hbm.at[pl.ds(bag0 + g * G_BAGS, G_BAGS)])
        return carry

    lax.fori_loop(0, N_GROUPS, group, 0)


_sc_mesh = plsc.VectorSubcoreMesh(
    core_axis_name="c", subcore_axis_name="s",
    num_cores=NUM_CORES, num_subcores=NUM_SUBCORES)

_sc_call = pl.kernel(
    _sc_body,
    out_type=jax.ShapeDtypeStruct((BATCH, EMBED), jnp.float32),
    mesh=_sc_mesh,
    scratch_types=[
        pltpu.VMEM((G_ROWS,), jnp.int32),
        pltpu.VMEM((G_ROWS, EMBED), jnp.float32),
        pltpu.VMEM((G_BAGS, EMBED), jnp.float32),
        pltpu.SemaphoreType.DMA,
    ],
    compiler_params=pltpu.CompilerParams(use_tc_tiling_on_sc=False),
)


def _tc_body(bag_ref, w_ref, b_ref, o_ref):
    x = lax.dot_general(bag_ref[...], w_ref[...],
                        (((1,), (1,)), ((), ())),
                        preferred_element_type=jnp.float32)
    x = x + b_ref[...]
    x = x - jnp.max(x, axis=-1, keepdims=True)
    e = jnp.exp(x)
    o_ref[...] = e / jnp.sum(e, axis=-1, keepdims=True)


def _tc_call(bag, W, b):
    BB = 2048
    return pl.pallas_call(
        _tc_body,
        grid=(BATCH // BB,),
        in_specs=[
            pl.BlockSpec((BB, EMBED), lambda i: (i, 0)),
            pl.BlockSpec((OUT_DIM, EMBED), lambda i: (0, 0)),
            pl.BlockSpec((1, OUT_DIM), lambda i: (0, 0)),
        ],
        out_specs=pl.BlockSpec((BB, OUT_DIM), lambda i: (i, 0)),
        out_shape=jax.ShapeDtypeStruct((BATCH, OUT_DIM), jnp.float32),
    )(bag, W, b.reshape(1, OUT_DIM))


def kernel(input, table, W, b):
    idx = input.reshape(-1).astype(jnp.int32)
    bag = _sc_call(idx, table)
    return _tc_call(bag, W, b)


# speedup vs baseline: 2.8660x; 1.0591x over previous
import jax
import jax.numpy as jnp
from jax import lax
from jax.experimental import pallas as pl
from jax.experimental.pallas import tpu as pltpu
from jax.experimental.pallas import tpu_sc as plsc

NUM_CORES = 2
NUM_SUBCORES = 16
NW = NUM_CORES * NUM_SUBCORES

BATCH = 16384
HIST = 50
EMBED = 32
OUT_DIM = 20

BAGS_PER_W = BATCH // NW
G_BAGS = 32
N_GROUPS = BAGS_PER_W // G_BAGS


def _sc_body(idx_hbm, table_hbm, out_hbm,
             idx_v, rows_v, out_v, isems, gsems, osems):
    wid = lax.axis_index("s") * NUM_CORES + lax.axis_index("c")
    bag0 = wid * BAGS_PER_W

    def stage_idx(g, slot):
        return pltpu.async_copy(
            idx_hbm.at[pl.ds(bag0 + g * G_BAGS, G_BAGS)],
            idx_v.at[slot], isems[slot])

    def fire_gather(p):
        return [pltpu.async_copy(
            table_hbm.at[idx_v.at[p, b]], rows_v.at[p, b], gsems[p])
            for b in range(G_BAGS)]

    def reduce_group(g, p):
        def bag(i, c):
            acc0 = jnp.zeros((16,), jnp.float32)
            acc1 = jnp.zeros((16,), jnp.float32)
            for r in range(HIST):
                acc0 = acc0 + rows_v[p, i, r, pl.ds(0, 16)]
                acc1 = acc1 + rows_v[p, i, r, pl.ds(16, 16)]
            out_v[p, i, pl.ds(0, 16)] = acc0 * (1.0 / HIST)
            out_v[p, i, pl.ds(16, 16)] = acc1 * (1.0 / HIST)
            return c
        lax.fori_loop(0, G_BAGS, bag, 0)
        return pltpu.async_copy(
            out_v.at[p],
            out_hbm.at[pl.ds(bag0 + g * G_BAGS, G_BAGS)], osems[p])

    h_idx = {0: stage_idx(0, 0)}
    h_gat = {}
    h_out = {}
    h_idx[0].wait()
    h_gat[0] = fire_gather(0)
    h_idx[1] = stage_idx(1, 1)
    for g in range(N_GROUPS):
        p = g % 2
        if g + 1 < N_GROUPS:
            h_idx[g + 1].wait()
            h_gat[g + 1] = fire_gather(1 - p)
        for h in h_gat[g]:
            h.wait()
        if g + 2 < N_GROUPS:
            h_idx[g + 2] = stage_idx(g + 2, p)
        if g - 2 >= 0:
            h_out[g - 2].wait()
        h_out[g] = reduce_group(g, p)
    h_out[N_GROUPS - 2].wait()
    h_out[N_GROUPS - 1].wait()


_sc_mesh = plsc.VectorSubcoreMesh(
    core_axis_name="c", subcore_axis_name="s",
    num_cores=NUM_CORES, num_subcores=NUM_SUBCORES)

_sc_call = pl.kernel(
    _sc_body,
    out_type=jax.ShapeDtypeStruct((BATCH, EMBED), jnp.float32),
    mesh=_sc_mesh,
    scratch_types=[
        pltpu.VMEM((2, G_BAGS, HIST), jnp.int32),
        pltpu.VMEM((2, G_BAGS, HIST, EMBED), jnp.float32),
        pltpu.VMEM((2, G_BAGS, EMBED), jnp.float32),
        [pltpu.SemaphoreType.DMA, pltpu.SemaphoreType.DMA],
        [pltpu.SemaphoreType.DMA, pltpu.SemaphoreType.DMA],
        [pltpu.SemaphoreType.DMA, pltpu.SemaphoreType.DMA],
    ],
    compiler_params=pltpu.CompilerParams(use_tc_tiling_on_sc=False),
)


def _tc_body(bag_ref, w_ref, b_ref, o_ref):
    x = lax.dot_general(bag_ref[...], w_ref[...],
                        (((1,), (1,)), ((), ())),
                        preferred_element_type=jnp.float32)
    x = x + b_ref[...]
    x = x - jnp.max(x, axis=-1, keepdims=True)
    e = jnp.exp(x)
    o_ref[...] = e / jnp.sum(e, axis=-1, keepdims=True)


def _tc_call(bag, W, b):
    BB = 2048
    return pl.pallas_call(
        _tc_body,
        grid=(BATCH // BB,),
        in_specs=[
            pl.BlockSpec((BB, EMBED), lambda i: (i, 0)),
            pl.BlockSpec((OUT_DIM, EMBED), lambda i: (0, 0)),
            pl.BlockSpec((1, OUT_DIM), lambda i: (0, 0)),
        ],
        out_specs=pl.BlockSpec((BB, OUT_DIM), lambda i: (i, 0)),
        out_shape=jax.ShapeDtypeStruct((BATCH, OUT_DIM), jnp.float32),
    )(bag, W, b.reshape(1, OUT_DIM))


def kernel(input, table, W, b):
    idx = input.astype(jnp.int32)
    bag = _sc_call(idx, table)
    return _tc_call(bag, W, b)
